# baseline (speedup 1.0000x reference)
import jax
import jax.numpy as jnp
from jax import lax
from jax.experimental import pallas as pl
from jax.experimental.pallas import tpu as pltpu
from jax.experimental.pallas import tpu_sc as plsc

_CH = 66
_D = 3


def _chunks(S):
    out = []
    r = 0
    while r < S:
        n = min(_CH, S - r)
        out.append((r, n))
        r += n
    return out


def _body(x, tgt, lens, c, sos, labels, x_out, tgt_out,
          buf, lens_s, c_s, *sems):
    B, S, J = x.shape
    T_L = labels.shape[1]
    gsem = sems[:_D]
    ssem = sems[_D:2 * _D]
    chunks = _chunks(S)
    NCH = len(chunks)

    wid = lax.axis_index("s") * 2 + lax.axis_index("c")
    b = wid % B
    kind = wid // B

    pltpu.sync_copy(lens, lens_s)
    pltpu.sync_copy(c, c_s)
    lane = lax.iota(jnp.int32, 16)
    lb = jnp.max(jnp.where(lane == b, lens_s[...], 0), axis=0)
    cb = jnp.max(jnp.where(lane == b, c_s[...], 0), axis=0)

    def gather(i, src):
        r, n = chunks[i]
        return pltpu.async_copy(
            src.at[b, pl.ds(r, n)],
            buf.at[i % _D, pl.ds(0, n)], gsem[i % _D])

    def scatter(i, dst, shift):
        r, n = chunks[i]
        return pltpu.async_copy(
            buf.at[i % _D, pl.ds(0, n)],
            dst.at[b, pl.ds(r + shift, n)], ssem[i % _D])

    def run(src, dst, shift, with_tail_sos):
        g = [None] * NCH
        s = [None] * NCH
        for i in range(min(_D, NCH)):
            g[i] = gather(i, src)
        for i in range(NCH):
            g[i].wait()
            s[i] = scatter(i, dst, shift)
            if i + _D < NCH:
                s[i].wait()
                g[i + _D] = gather(i + _D, src)
        for i in range(max(0, NCH - _D), NCH):
            s[i].wait()

        wrows = T_L + 2 if with_tail_sos else T_L + 1
        wd = [
            pltpu.async_copy(sos.at[pl.ds(b, 1)], buf.at[0, pl.ds(0, 1)], gsem[0]),
            pltpu.async_copy(labels.at[cb], buf.at[0, pl.ds(1, T_L)], gsem[1]),
        ]
        if with_tail_sos:
            wd.append(pltpu.async_copy(sos.at[pl.ds(b, 1)],
                                       buf.at[0, pl.ds(T_L + 1, 1)], gsem[2]))
        else:
            wd.append(pltpu.async_copy(sos.at[pl.ds(b, 1)],
                                       dst.at[b, pl.ds(0, 1)], gsem[2]))
        for d in wd:
            d.wait()
        pltpu.sync_copy(buf.at[0, pl.ds(0, wrows)],
                        dst.at[b, pl.ds(lb + shift, wrows)])

    @pl.when(kind == 0)
    def _():
        run(x, x_out, 1, False)

    @pl.when(kind == 1)
    def _():
        run(tgt, tgt_out, 0, True)


def kernel(x, tgt, lens, c, sos, labels):
    B, S, J = x.shape
    run = pl.kernel(
        _body,
        out_type=(
            jax.ShapeDtypeStruct((B, S + 1, J), x.dtype),
            jax.ShapeDtypeStruct((B, S, J), tgt.dtype),
        ),
        mesh=plsc.VectorSubcoreMesh(core_axis_name="c", subcore_axis_name="s"),
        compiler_params=pltpu.CompilerParams(
            use_tc_tiling_on_sc=False, needs_layout_passes=False
        ),
        scratch_types=[
            pltpu.VMEM((_D, _CH, J), x.dtype),
            pltpu.VMEM((B,), jnp.int32),
            pltpu.VMEM((B,), jnp.int32),
        ] + [pltpu.SemaphoreType.DMA] * (2 * _D),
    )
    x_out, tgt_out = run(x, tgt, lens, c, sos, labels)
    return (x_out, tgt_out, labels)
